# baseline (speedup 1.0000x reference)
import jax
import jax.numpy as jnp
from jax import lax
from jax.experimental import pallas as pl
from jax.experimental.pallas import tpu as pltpu
from jax.experimental.pallas import tpu_sc as plsc

_B = 16384
_F = 26
_D = 32
_L = 16
_NW = 32
_RPW = _B // _NW
_CHUNK = 64
_NCHUNK = _RPW // _CHUNK
_GPC = _CHUNK // _L
_IPC = _CHUNK * _F
_NSTREAM = _IPC // 128


def _body(feat_hbm, fvt_hbm, emb_hbm, bias_hbm, out_hbm,
          idx_v, emb_v, bias_v, fv_v, out_v, sem_e, sem_b):
    nc = 2
    wid = lax.axis_index("s") * nc + lax.axis_index("c")
    iota = lax.iota(jnp.int32, _L)
    iota_f = iota * _F
    iota_fd = iota * (_F * _D)
    zero_v = jnp.zeros((_L,), jnp.float32)
    zeros_i = jnp.zeros((_L,), jnp.int32)

    @pl.loop(0, _NCHUNK)
    def _chunk(c):
        i0 = wid * (_NCHUNK * _IPC) + c * _IPC
        pltpu.sync_copy(feat_hbm.at[pl.ds(i0, _IPC)], idx_v)
        g0 = wid * (_NCHUNK * _GPC) + c * _GPC
        pltpu.sync_copy(fvt_hbm.at[pl.ds(g0 * (_F * _L), _GPC * _F * _L)],
                        fv_v)
        descs = []
        for j in range(_NSTREAM):
            descs.append(pltpu.async_copy(
                emb_hbm.at[idx_v.at[pl.ds(j * 128, 128)]],
                emb_v.at[pl.ds(j * 128, 128), :], sem_e))
            descs.append(pltpu.async_copy(
                bias_hbm.at[idx_v.at[pl.ds(j * 128, 128)]],
                bias_v.at[pl.ds(j * 128, 128), :], sem_b))
        for d in descs:
            d.wait()
        for g in range(_GPC):
            gbase = iota_f + g * (_L * _F)
            total = zero_v
            for half in range(2):
                def fbody(f, carry, _half=half, _gbase=gbase, _g=g):
                    rvec = _gbase + f
                    vvec = fv_v[pl.ds((_g * _F + f) * _L, _L)]
                    sums = []
                    sqs = []
                    for d in range(_L):
                        col = jnp.full((_L,), _half * _L + d, jnp.int32)
                        e = plsc.load_gather(emb_v, [rvec, col])
                        ev = e * vvec
                        sums.append(carry[d] + ev)
                        sqs.append(carry[_L + d] + ev * ev)
                    return tuple(sums) + tuple(sqs)

                accs = lax.fori_loop(0, _F, fbody, (zero_v,) * (2 * _L))
                for d in range(_L):
                    a = accs[d]
                    total = total + (a * a - accs[_L + d])
            total = total * 0.5

            def bbody(f, bt, _gbase=gbase, _g=g):
                bv = plsc.load_gather(bias_v, [_gbase + f, zeros_i])
                vvec = fv_v[pl.ds((_g * _F + f) * _L, _L)]
                return bt + bv * vvec

            total = total + lax.fori_loop(0, _F, bbody, zero_v)
            out_v[pl.ds(g * _L, _L)] = total
        pltpu.sync_copy(out_v, out_hbm.at[pl.ds(wid * _RPW + c * _CHUNK,
                                                _CHUNK)])


@jax.jit
def _pointfm_sc(feat2d, fvt, emb_table, bias_table):
    mesh = plsc.VectorSubcoreMesh(core_axis_name="c", subcore_axis_name="s")
    return pl.kernel(
        _body,
        out_type=jax.ShapeDtypeStruct((_B,), jnp.float32),
        mesh=mesh,
        scratch_types=[
            pltpu.VMEM((_IPC,), jnp.int32),
            pltpu.VMEM((_IPC, _D), jnp.float32),
            pltpu.VMEM((_IPC, 1), jnp.float32),
            pltpu.VMEM((_GPC * _F * _L,), jnp.float32),
            pltpu.VMEM((_CHUNK,), jnp.float32),
            pltpu.SemaphoreType.DMA,
            pltpu.SemaphoreType.DMA,
        ],
        compiler_params=pltpu.CompilerParams(
            needs_layout_passes=False, use_tc_tiling_on_sc=False),
    )(feat2d, fvt, emb_table, bias_table)


def kernel(features, feature_values, emb_table, bias_table, bias_):
    feat2d = features.reshape(-1)
    fvt = jnp.transpose(
        feature_values.reshape(_B // _L, _L, _F), (0, 2, 1)).reshape(-1)
    out = _pointfm_sc(feat2d, fvt, emb_table, bias_table)
    return out + bias_


# speedup vs baseline: 1.1040x; 1.1040x over previous
import jax
import jax.numpy as jnp
from jax import lax
from jax.experimental import pallas as pl
from jax.experimental.pallas import tpu as pltpu
from jax.experimental.pallas import tpu_sc as plsc

_B = 16384
_F = 26
_D = 32
_L = 16
_NW = 32
_RPW = _B // _NW
_CHUNK = 64
_NCHUNK = _RPW // _CHUNK
_IPC = _CHUNK * _F
_NSTREAM = _IPC // 128


def _body(feat_hbm, fv_hbm, emb_hbm, bias_hbm, out_hbm,
          idx_v, emb_v, bias_v, fv_v, out_v, sem_e, sem_b):
    nc = 2
    wid = lax.axis_index("s") * nc + lax.axis_index("c")
    iota = lax.iota(jnp.int32, _L)
    m1 = iota < (_F - _L)
    mlast = iota == (_L - 1)
    zeros_i = jnp.zeros((_L,), jnp.int32)
    zero_v = jnp.zeros((_L,), jnp.float32)

    @pl.loop(0, _NCHUNK)
    def _chunk(c):
        i0 = wid * (_NCHUNK * _IPC) + c * _IPC
        pltpu.sync_copy(feat_hbm.at[pl.ds(i0, _IPC)], idx_v)
        pltpu.sync_copy(fv_hbm.at[pl.ds(i0, _IPC)], fv_v)
        descs = []
        for j in range(_NSTREAM):
            descs.append(pltpu.async_copy(
                emb_hbm.at[idx_v.at[pl.ds(j * 128, 128)]],
                emb_v.at[pl.ds(j * 128, 128), :], sem_e))
            descs.append(pltpu.async_copy(
                bias_hbm.at[idx_v.at[pl.ds(j * 128, 128)]],
                bias_v.at[pl.ds(j * 128, 128), :], sem_b))
        for d in descs:
            d.wait()

        @pl.loop(0, _CHUNK)
        def _row(b):
            r0 = b * _F
            ix0 = r0 + iota
            ix1 = ix0 + _L
            vv0 = plsc.load_gather(fv_v, [ix0])
            vv1 = plsc.load_gather(fv_v, [ix1], mask=m1)
            acc0 = zero_v
            acc1 = zero_v
            sq0 = zero_v
            sq1 = zero_v
            rsplat = jnp.full((_L,), 0, jnp.int32) + r0
            for f in range(_F):
                sv = (vv0 if f < _L else vv1)[f % _L]
                rvec = rsplat + f
                e0 = plsc.load_gather(emb_v, [rvec, iota])
                e1 = plsc.load_gather(emb_v, [rvec, iota + _L])
                ev0 = e0 * sv
                ev1 = e1 * sv
                acc0 = acc0 + ev0
                acc1 = acc1 + ev1
                sq0 = sq0 + ev0 * ev0
                sq1 = sq1 + ev1 * ev1
            fm = acc0 * acc0 - sq0 + acc1 * acc1 - sq1
            bb0 = plsc.load_gather(bias_v, [ix0, zeros_i])
            bb1 = plsc.load_gather(bias_v, [ix1, zeros_i], mask=m1)
            bvec = bb0 * vv0 + jnp.where(m1, bb1 * vv1, 0.0)
            t = lax.reduce_sum(0.5 * fm + bvec, axes=(0,))
            tv = jnp.full((_L,), 0.0, jnp.float32) + t
            plsc.store_scatter(out_v, [jnp.full((_L,), 0, jnp.int32) + b],
                               tv, mask=mlast)

        pltpu.sync_copy(out_v, out_hbm.at[pl.ds(wid * _RPW + c * _CHUNK,
                                                _CHUNK)])


@jax.jit
def _pointfm_sc(feat_flat, fv_flat, emb_table, bias_table):
    mesh = plsc.VectorSubcoreMesh(core_axis_name="c", subcore_axis_name="s")
    return pl.kernel(
        _body,
        out_type=jax.ShapeDtypeStruct((_B,), jnp.float32),
        mesh=mesh,
        scratch_types=[
            pltpu.VMEM((_IPC,), jnp.int32),
            pltpu.VMEM((_IPC, _D), jnp.float32),
            pltpu.VMEM((_IPC, 1), jnp.float32),
            pltpu.VMEM((_IPC,), jnp.float32),
            pltpu.VMEM((_CHUNK,), jnp.float32),
            pltpu.SemaphoreType.DMA,
            pltpu.SemaphoreType.DMA,
        ],
        compiler_params=pltpu.CompilerParams(
            needs_layout_passes=False, use_tc_tiling_on_sc=False),
    )(feat_flat, fv_flat, emb_table, bias_table)


def kernel(features, feature_values, emb_table, bias_table, bias_):
    out = _pointfm_sc(features.reshape(-1), feature_values.reshape(-1),
                      emb_table, bias_table)
    return out + bias_


# speedup vs baseline: 2.2497x; 2.0377x over previous
import jax
import jax.numpy as jnp
from jax import lax
from jax.experimental import pallas as pl
from jax.experimental.pallas import tpu as pltpu
from jax.experimental.pallas import tpu_sc as plsc

_B = 16384
_F = 26
_D = 32
_L = 16
_NW = 32
_RPW = _B // _NW
_CHUNK = 16
_NCHUNK = _RPW // _CHUNK
_IPC = _CHUNK * _F
_SROW = 128


def _body(feat_hbm, fv_hbm, emb_hbm, bias_hbm, out_hbm,
          idx_v, idxq_v, emb_v, bias_v, fv_v, out_v, sem_e, sem_b):
    nc = 2
    wid = lax.axis_index("s") * nc + lax.axis_index("c")
    iota = lax.iota(jnp.int32, _L)
    m1 = iota < (_F - _L)
    mlast = iota == (_L - 1)
    zero_v = jnp.zeros((_L,), jnp.float32)

    @pl.loop(0, _NCHUNK)
    def _chunk(c):
        i0 = wid * (_NCHUNK * _IPC) + c * _IPC
        pltpu.sync_copy(feat_hbm.at[pl.ds(i0, _IPC)], idx_v)
        pltpu.sync_copy(fv_hbm.at[pl.ds(i0, _IPC)], fv_v)
        for j in range(_IPC // _L):
            idxq_v[pl.ds(j * _L, _L)] = (
                lax.shift_right_logical(idx_v[pl.ds(j * _L, _L)], 2))
        descs = []
        for j0 in range(0, _IPC, 128):
            n = min(128, _IPC - j0)
            descs.append(pltpu.async_copy(
                emb_hbm.at[idxq_v.at[pl.ds(j0, n)]],
                emb_v.at[pl.ds(j0, n), :], sem_e))
            descs.append(pltpu.async_copy(
                bias_hbm.at[idx_v.at[pl.ds(j0, n)]],
                bias_v.at[pl.ds(j0, n)], sem_b))
        for d in descs:
            d.wait()

        @pl.loop(0, _CHUNK)
        def _row(b):
            r0 = b * _F
            ix0 = r0 + iota
            ix1 = ix0 + _L
            vv0 = plsc.load_gather(fv_v, [ix0])
            vv1 = plsc.load_gather(fv_v, [ix1], mask=m1)
            qo0 = (plsc.load_gather(idx_v, [ix0]) & 3) * _D
            qo1 = (plsc.load_gather(idx_v, [ix1], mask=m1) & 3) * _D
            acc0 = zero_v
            acc1 = zero_v
            sq0 = zero_v
            sq1 = zero_v
            rsplat = jnp.full((_L,), 0, jnp.int32) + r0
            for f in range(_F):
                sv = (vv0 if f < _L else vv1)[f % _L]
                co = (qo0 if f < _L else qo1)[f % _L]
                rvec = rsplat + f
                c0 = co + iota
                e0 = plsc.load_gather(emb_v, [rvec, c0])
                e1 = plsc.load_gather(emb_v, [rvec, c0 + _L])
                ev0 = e0 * sv
                ev1 = e1 * sv
                acc0 = acc0 + ev0
                acc1 = acc1 + ev1
                sq0 = sq0 + ev0 * ev0
                sq1 = sq1 + ev1 * ev1
            fm = acc0 * acc0 - sq0 + acc1 * acc1 - sq1
            bb0 = plsc.load_gather(bias_v, [ix0])
            bb1 = plsc.load_gather(bias_v, [ix1], mask=m1)
            bvec = bb0 * vv0 + jnp.where(m1, bb1 * vv1, 0.0)
            t = lax.reduce_sum(0.5 * fm + bvec, axes=(0,))
            tv = jnp.full((_L,), 0.0, jnp.float32) + t
            plsc.store_scatter(out_v, [jnp.full((_L,), 0, jnp.int32) + b],
                               tv, mask=mlast)

        pltpu.sync_copy(out_v, out_hbm.at[pl.ds(wid * _RPW + c * _CHUNK,
                                                _CHUNK)])


@jax.jit
def _pointfm_sc(feat_flat, fv_flat, emb2, bias_flat):
    mesh = plsc.VectorSubcoreMesh(core_axis_name="c", subcore_axis_name="s")
    return pl.kernel(
        _body,
        out_type=jax.ShapeDtypeStruct((_B,), jnp.float32),
        mesh=mesh,
        scratch_types=[
            pltpu.VMEM((_IPC,), jnp.int32),
            pltpu.VMEM((_IPC,), jnp.int32),
            pltpu.VMEM((_IPC, _SROW), jnp.float32),
            pltpu.VMEM((_IPC,), jnp.float32),
            pltpu.VMEM((_IPC,), jnp.float32),
            pltpu.VMEM((_CHUNK,), jnp.float32),
            pltpu.SemaphoreType.DMA,
            pltpu.SemaphoreType.DMA,
        ],
        compiler_params=pltpu.CompilerParams(
            needs_layout_passes=False, use_tc_tiling_on_sc=False),
    )(feat_flat, fv_flat, emb2, bias_flat)


def kernel(features, feature_values, emb_table, bias_table, bias_):
    out = _pointfm_sc(features.reshape(-1), feature_values.reshape(-1),
                      emb_table.reshape(-1, _SROW), bias_table.reshape(-1))
    return out + bias_


# speedup vs baseline: 2.4306x; 1.0804x over previous
import jax
import jax.numpy as jnp
from jax import lax
from jax.experimental import pallas as pl
from jax.experimental.pallas import tpu as pltpu
from jax.experimental.pallas import tpu_sc as plsc

_B = 16384
_F = 26
_D = 32
_L = 16
_NW = 32
_RPW = _B // _NW
_CHUNK = 16
_NCHUNK = _RPW // _CHUNK
_IPC = _CHUNK * _F
_SROW = 128


def _body(feat_hbm, fv_hbm, emb_hbm, bias_hbm, out_hbm,
          idx_v, emb_v, bias_v, fv_v, out_v, sem_e, sem_b):
    nc = 2
    wid = lax.axis_index("s") * nc + lax.axis_index("c")
    iota = lax.iota(jnp.int32, _L)
    m1 = iota < (_F - _L)
    mlast = iota == (_L - 1)
    zero_v = jnp.zeros((_L,), jnp.float32)

    @pl.loop(0, _NCHUNK)
    def _chunk(c):
        i0 = wid * (_NCHUNK * _IPC) + c * _IPC
        pltpu.sync_copy(feat_hbm.at[pl.ds(i0, _IPC)], idx_v)
        pltpu.sync_copy(fv_hbm.at[pl.ds(i0, _IPC)], fv_v)
        descs = []
        for j0 in range(0, _IPC, 128):
            n = min(128, _IPC - j0)
            descs.append(pltpu.async_copy(
                emb_hbm.at[idx_v.at[pl.ds(j0, n)]],
                emb_v.at[pl.ds(j0, n), :], sem_e))
            descs.append(pltpu.async_copy(
                bias_hbm.at[idx_v.at[pl.ds(j0, n)]],
                bias_v.at[pl.ds(j0, n)], sem_b))
        for d in descs:
            d.wait()

        @pl.loop(0, _CHUNK)
        def _row(b):
            r0 = b * _F
            ix0 = r0 + iota
            ix1 = ix0 + _L
            vv0 = plsc.load_gather(fv_v, [ix0])
            vv1 = plsc.load_gather(fv_v, [ix1], mask=m1)
            acc0 = zero_v
            acc1 = zero_v
            sq0 = zero_v
            sq1 = zero_v
            rsplat = jnp.full((_L,), 0, jnp.int32) + r0
            for f in range(_F):
                sv = (vv0 if f < _L else vv1)[f % _L]
                rvec = rsplat + f
                e0 = plsc.load_gather(emb_v, [rvec, iota])
                e1 = plsc.load_gather(emb_v, [rvec, iota + _L])
                ev0 = e0 * sv
                ev1 = e1 * sv
                acc0 = acc0 + ev0
                acc1 = acc1 + ev1
                sq0 = sq0 + ev0 * ev0
                sq1 = sq1 + ev1 * ev1
            fm = acc0 * acc0 - sq0 + acc1 * acc1 - sq1
            bb0 = plsc.load_gather(bias_v, [ix0])
            bb1 = plsc.load_gather(bias_v, [ix1], mask=m1)
            bvec = bb0 * vv0 + jnp.where(m1, bb1 * vv1, 0.0)
            t = lax.reduce_sum(0.5 * fm + bvec, axes=(0,))
            tv = jnp.full((_L,), 0.0, jnp.float32) + t
            plsc.store_scatter(out_v, [jnp.full((_L,), 0, jnp.int32) + b],
                               tv, mask=mlast)

        pltpu.sync_copy(out_v, out_hbm.at[pl.ds(wid * _RPW + c * _CHUNK,
                                                _CHUNK)])


@jax.jit
def _pointfm_sc(feat_flat, fv_flat, emb2, bias_flat):
    mesh = plsc.VectorSubcoreMesh(core_axis_name="c", subcore_axis_name="s")
    return pl.kernel(
        _body,
        out_type=jax.ShapeDtypeStruct((_B,), jnp.float32),
        mesh=mesh,
        scratch_types=[
            pltpu.VMEM((_IPC,), jnp.int32),
            pltpu.VMEM((_IPC, _D), jnp.float32),
            pltpu.VMEM((_IPC,), jnp.float32),
            pltpu.VMEM((_IPC,), jnp.float32),
            pltpu.VMEM((_CHUNK,), jnp.float32),
            pltpu.SemaphoreType.DMA,
            pltpu.SemaphoreType.DMA,
        ],
        compiler_params=pltpu.CompilerParams(
            needs_layout_passes=False, use_tc_tiling_on_sc=False),
    )(feat_flat, fv_flat, emb2, bias_flat)


def kernel(features, feature_values, emb_table, bias_table, bias_):
    out = _pointfm_sc(features.reshape(-1), feature_values.reshape(-1),
                      emb_table, bias_table.reshape(-1))
    return out + bias_


# speedup vs baseline: 3.2822x; 1.3504x over previous
import numpy as np

import jax
import jax.numpy as jnp
from jax import lax
from jax.experimental import pallas as pl
from jax.experimental.pallas import tpu as pltpu
from jax.experimental.pallas import tpu_sc as plsc

_B = 16384
_F = 26
_D = 32
_V = 1000000
_L = 16
_NW = 32
_RPW = _B // _NW
_CHUNK = 16
_NCHUNK = _RPW // _CHUNK
_IPC = _CHUNK * _F
_SROW = 128
_NSUP = _V * _D // _SROW
_NBLK = _V // _SROW
_KPT = _NBLK // _NW

def _tbody(embt_hbm, tail_hbm, sup_hbm, in_v, out_v,
           sem_i0, sem_i1, sem_o0, sem_o1):
    nc = 2
    wid = lax.axis_index("s") * nc + lax.axis_index("c")
    iota = lax.iota(jnp.int32, _L)
    basek, orowc, ocolc = [], [], []
    for k in range(16):
        cv = (iota + k) & 15
        basek.append(cv)
        orow_k, ocol_k = [], []
        for dh in range(2):
            flat = cv * _D + (iota + dh * 16)
            orow_k.append(lax.shift_right_logical(flat, 7))
            ocol_k.append(flat & 127)
        orowc.append(orow_k)
        ocolc.append(ocol_k)
    sems_i = (sem_i0, sem_i1)
    sems_o = (sem_o0, sem_o1)
    slotc = (jnp.zeros((_L,), jnp.int32), jnp.zeros((_L,), jnp.int32) + 1)

    def issue_in(slot, blk):
        for dg in range(4):
            pltpu.async_copy(
                embt_hbm.at[pl.ds(dg * 8, 8), pl.ds(blk * _SROW, _SROW)],
                in_v.at[slot, pl.ds(dg * 8, 8), :], sems_i[slot])

    def wait_in(slot):
        for dg in range(4):
            pltpu.make_async_copy(
                embt_hbm.at[pl.ds(0, 8), pl.ds(0, _SROW)],
                in_v.at[slot, pl.ds(dg * 8, 8), :], sems_i[slot]).wait()

    def compute(slot):
        @pl.loop(0, 8)
        def _isb(isb):
            isb16 = isb * 16
            isb4 = isb * 4
            for dh in range(2):
                rv = iota + dh * 16
                for k in range(16):
                    cvec = basek[k] + isb16
                    orow = orowc[k][dh] + isb4
                    v = plsc.load_gather(in_v, [slotc[slot], rv, cvec])
                    plsc.store_scatter(
                        out_v, [slotc[slot], orow, ocolc[k][dh]], v)

    def issue_out(slot, blk, nrow=32):
        pltpu.async_copy(out_v.at[slot, pl.ds(0, nrow), :],
                         sup_hbm.at[pl.ds(blk * 32, nrow), :], sems_o[slot])

    def wait_out(slot, nrow=32):
        pltpu.make_async_copy(out_v.at[slot, pl.ds(0, nrow), :],
                              sup_hbm.at[pl.ds(0, nrow), :],
                              sems_o[slot]).wait()

    issue_in(0, wid)
    issue_in(1, wid + _NW)

    @pl.loop(0, _KPT // 2)
    def _pair(kk):
        for s in range(2):
            k = kk * 2 + s
            blk = wid + k * _NW
            wait_in(s)

            @pl.when(k >= 2)
            def _():
                wait_out(s)

            compute(s)
            issue_out(s, blk)

            @pl.when(k + 2 < _KPT)
            def _():
                issue_in(s, wid + (k + 2) * _NW)

    wait_out(0)
    wait_out(1)

    @pl.when(wid < 4)
    def _extra():
        blk = _NW * _KPT + wid
        issue_in(0, blk)
        wait_in(0)
        compute(0)
        issue_out(0, blk)
        wait_out(0)

    @pl.when(wid == 4)
    def _tail():
        pltpu.sync_copy(tail_hbm, in_v.at[1, pl.ds(0, 16), :])
        pltpu.sync_copy(in_v.at[1, pl.ds(0, 16), :],
                        sup_hbm.at[pl.ds(_NSUP - 16, 16), :])


def _gbody(feat_hbm, fv_hbm, sup_hbm, bias_hbm, out_hbm,
           idx_v, idxq_v, emb_v, bias_v, fv_v, out_v, sem_e, sem_b):
    nc = 2
    wid = lax.axis_index("s") * nc + lax.axis_index("c")
    iota = lax.iota(jnp.int32, _L)
    m1 = iota < (_F - _L)
    mlast = iota == (_L - 1)
    zero_v = jnp.zeros((_L,), jnp.float32)

    @pl.loop(0, _NCHUNK)
    def _chunk(c):
        i0 = wid * (_NCHUNK * _IPC) + c * _IPC
        pltpu.sync_copy(feat_hbm.at[pl.ds(i0, _IPC)], idx_v)
        pltpu.sync_copy(fv_hbm.at[pl.ds(i0, _IPC)], fv_v)
        for j in range(_IPC // _L):
            idxq_v[pl.ds(j * _L, _L)] = (
                lax.shift_right_logical(idx_v[pl.ds(j * _L, _L)], 2))
        descs = []
        for j0 in range(0, _IPC, 128):
            n = min(128, _IPC - j0)
            descs.append(pltpu.async_copy(
                sup_hbm.at[idxq_v.at[pl.ds(j0, n)]],
                emb_v.at[pl.ds(j0, n), :], sem_e))
            descs.append(pltpu.async_copy(
                bias_hbm.at[idx_v.at[pl.ds(j0, n)]],
                bias_v.at[pl.ds(j0, n)], sem_b))
        for d in descs:
            d.wait()

        @pl.loop(0, _CHUNK)
        def _row(b):
            r0 = b * _F
            ix0 = r0 + iota
            ix1 = ix0 + _L
            vv0 = plsc.load_gather(fv_v, [ix0])
            vv1 = plsc.load_gather(fv_v, [ix1], mask=m1)
            qo0 = (plsc.load_gather(idx_v, [ix0]) & 3) * _D
            qo1 = (plsc.load_gather(idx_v, [ix1], mask=m1) & 3) * _D
            acc0 = zero_v
            acc1 = zero_v
            sq0 = zero_v
            sq1 = zero_v
            rsplat = jnp.full((_L,), 0, jnp.int32) + r0
            for f in range(_F):
                sv = (vv0 if f < _L else vv1)[f % _L]
                co = (qo0 if f < _L else qo1)[f % _L]
                rvec = rsplat + f
                c0 = co + iota
                e0 = plsc.load_gather(emb_v, [rvec, c0])
                e1 = plsc.load_gather(emb_v, [rvec, c0 + _L])
                ev0 = e0 * sv
                ev1 = e1 * sv
                acc0 = acc0 + ev0
                acc1 = acc1 + ev1
                sq0 = sq0 + ev0 * ev0
                sq1 = sq1 + ev1 * ev1
            fm = acc0 * acc0 - sq0 + acc1 * acc1 - sq1
            bb0 = plsc.load_gather(bias_v, [ix0])
            bb1 = plsc.load_gather(bias_v, [ix1], mask=m1)
            bvec = bb0 * vv0 + jnp.where(m1, bb1 * vv1, 0.0)
            t = lax.reduce_sum(0.5 * fm + bvec, axes=(0,))
            tv = jnp.full((_L,), 0.0, jnp.float32) + t
            plsc.store_scatter(out_v, [jnp.full((_L,), 0, jnp.int32) + b],
                               tv, mask=mlast)

        pltpu.sync_copy(out_v, out_hbm.at[pl.ds(wid * _RPW + c * _CHUNK,
                                                _CHUNK)])


_MESH = plsc.VectorSubcoreMesh(core_axis_name="c", subcore_axis_name="s")
_PARAMS = pltpu.CompilerParams(
    needs_layout_passes=False, use_tc_tiling_on_sc=True)


@jax.jit
def _pointfm_sc(feat_flat, fv_flat, emb_t, tail16, bias_flat):
    sup = pl.kernel(
        _tbody,
        out_type=jax.ShapeDtypeStruct((_NSUP, _SROW), jnp.float32),
        mesh=_MESH,
        scratch_types=[
            pltpu.VMEM((2, _D, _SROW), jnp.float32),
            pltpu.VMEM((2, _D, _SROW), jnp.float32),
            pltpu.SemaphoreType.DMA,
            pltpu.SemaphoreType.DMA,
            pltpu.SemaphoreType.DMA,
            pltpu.SemaphoreType.DMA,
        ],
        compiler_params=_PARAMS,
    )(emb_t, tail16)
    return pl.kernel(
        _gbody,
        out_type=jax.ShapeDtypeStruct((_B,), jnp.float32),
        mesh=_MESH,
        scratch_types=[
            pltpu.VMEM((_IPC,), jnp.int32),
            pltpu.VMEM((_IPC,), jnp.int32),
            pltpu.VMEM((_IPC, _SROW), jnp.float32),
            pltpu.VMEM((_IPC,), jnp.float32),
            pltpu.VMEM((_IPC,), jnp.float32),
            pltpu.VMEM((_CHUNK,), jnp.float32),
            pltpu.SemaphoreType.DMA,
            pltpu.SemaphoreType.DMA,
        ],
        compiler_params=_PARAMS,
    )(feat_flat, fv_flat, sup, bias_flat)


def kernel(features, feature_values, emb_table, bias_table, bias_):
    tail16 = emb_table[_V - 2 * _D:].reshape(16, _SROW)
    out = _pointfm_sc(features.reshape(-1), feature_values.reshape(-1),
                      emb_table.T, tail16, bias_table.reshape(-1))
    return out + bias_


# speedup vs baseline: 3.3310x; 1.0149x over previous
import numpy as np

import jax
import jax.numpy as jnp
from jax import lax
from jax.experimental import pallas as pl
from jax.experimental.pallas import tpu as pltpu
from jax.experimental.pallas import tpu_sc as plsc

_B = 16384
_F = 26
_D = 32
_V = 1000000
_L = 16
_NW = 32
_RPW = _B // _NW
_CHUNK = 16
_NCHUNK = _RPW // _CHUNK
_IPC = _CHUNK * _F
_SROW = 128
_NSUP = _V * _D // _SROW
_NBLK = _V // _SROW
_W = 512
_NSB = _V // _W - 1
_KPT = _NSB // _NW

def _tbody(embt_hbm, tail_hbm, sup_hbm, in_v, out_v,
           sem_i0, sem_i1, sem_o0, sem_o1):
    nc = 2
    wid = lax.axis_index("s") * nc + lax.axis_index("c")
    iota = lax.iota(jnp.int32, _L)
    basek, orowc, ocolc = [], [], []
    for k in range(16):
        cv = (iota + k) & 15
        basek.append(cv)
        orow_k, ocol_k = [], []
        for dh in range(2):
            flat = cv * _D + (iota + dh * 16)
            orow_k.append(lax.shift_right_logical(flat, 7))
            ocol_k.append(flat & 127)
        orowc.append(orow_k)
        ocolc.append(ocol_k)
    sems_i = (sem_i0, sem_i1)
    sems_o = (sem_o0, sem_o1)
    slotc = (jnp.zeros((_L,), jnp.int32), jnp.zeros((_L,), jnp.int32) + 1)

    def issue_in(slot, sb, w=_W):
        for dg in range(4):
            pltpu.async_copy(
                embt_hbm.at[pl.ds(dg * 8, 8), pl.ds(sb * _W, w)],
                in_v.at[slot, pl.ds(dg * 8, 8), pl.ds(0, w)], sems_i[slot])

    def wait_in(slot, w=_W):
        for dg in range(4):
            pltpu.make_async_copy(
                embt_hbm.at[pl.ds(0, 8), pl.ds(0, w)],
                in_v.at[slot, pl.ds(dg * 8, 8), pl.ds(0, w)],
                sems_i[slot]).wait()

    def compute(slot, nsb=_W // _L):
        @pl.loop(0, nsb)
        def _isb(isb):
            isb16 = isb * 16
            isb4 = isb * 4
            for dh in range(2):
                rv = iota + dh * 16
                for k in range(16):
                    cvec = basek[k] + isb16
                    orow = orowc[k][dh] + isb4
                    v = plsc.load_gather(in_v, [slotc[slot], rv, cvec])
                    plsc.store_scatter(
                        out_v, [slotc[slot], orow, ocolc[k][dh]], v)

    def issue_out(slot, srow0, nrow=_W // 4):
        pltpu.async_copy(out_v.at[slot, pl.ds(0, nrow), :],
                         sup_hbm.at[pl.ds(srow0, nrow), :], sems_o[slot])

    def wait_out(slot, nrow=_W // 4):
        pltpu.make_async_copy(out_v.at[slot, pl.ds(0, nrow), :],
                              sup_hbm.at[pl.ds(0, nrow), :],
                              sems_o[slot]).wait()

    issue_in(0, wid)
    issue_in(1, wid + _NW)

    @pl.loop(0, (_KPT + 1) // 2)
    def _pair(kk):
        for s in range(2):
            k = kk * 2 + s

            @pl.when(k < _KPT)
            def _():
                sb = wid + k * _NW
                wait_in(s)

                @pl.when(k >= 2)
                def _():
                    wait_out(s)

                compute(s)
                issue_out(s, sb * (_W // 4))

                @pl.when(k + 2 < _KPT)
                def _():
                    issue_in(s, wid + (k + 2) * _NW)

    wait_out(0)
    wait_out(1)

    @pl.when(wid < 4)
    def _extra():
        i0 = _NSB * _W // _SROW + wid
        for dg in range(4):
            pltpu.async_copy(
                embt_hbm.at[pl.ds(dg * 8, 8), pl.ds(i0 * _SROW, _SROW)],
                in_v.at[0, pl.ds(dg * 8, 8), pl.ds(0, _SROW)], sem_i0)
        for dg in range(4):
            pltpu.make_async_copy(
                embt_hbm.at[pl.ds(0, 8), pl.ds(0, _SROW)],
                in_v.at[0, pl.ds(dg * 8, 8), pl.ds(0, _SROW)],
                sem_i0).wait()
        compute(0, nsb=_SROW // _L)
        issue_out(0, i0 * 32, nrow=32)
        wait_out(0, nrow=32)

    @pl.when(wid == 4)
    def _tail():
        pltpu.sync_copy(tail_hbm, in_v.at[1, pl.ds(0, 16), pl.ds(0, _SROW)])
        pltpu.sync_copy(in_v.at[1, pl.ds(0, 16), pl.ds(0, _SROW)],
                        sup_hbm.at[pl.ds(_NSUP - 16, 16), :])


def _gbody(feat_hbm, fv_hbm, sup_hbm, bias_hbm, out_hbm,
           idx_v, idxq_v, emb_v, bias_v, fv_v, out_v, sem_e, sem_b):
    nc = 2
    wid = lax.axis_index("s") * nc + lax.axis_index("c")
    iota = lax.iota(jnp.int32, _L)
    m1 = iota < (_F - _L)
    mlast = iota == (_L - 1)
    zero_v = jnp.zeros((_L,), jnp.float32)

    @pl.loop(0, _NCHUNK)
    def _chunk(c):
        i0 = wid * (_NCHUNK * _IPC) + c * _IPC
        pltpu.sync_copy(feat_hbm.at[pl.ds(i0, _IPC)], idx_v)
        pltpu.sync_copy(fv_hbm.at[pl.ds(i0, _IPC)], fv_v)
        for j in range(_IPC // _L):
            idxq_v[pl.ds(j * _L, _L)] = (
                lax.shift_right_logical(idx_v[pl.ds(j * _L, _L)], 2))
        descs = []
        for j0 in range(0, _IPC, 128):
            n = min(128, _IPC - j0)
            descs.append(pltpu.async_copy(
                sup_hbm.at[idxq_v.at[pl.ds(j0, n)]],
                emb_v.at[pl.ds(j0, n), :], sem_e))
            descs.append(pltpu.async_copy(
                bias_hbm.at[idx_v.at[pl.ds(j0, n)]],
                bias_v.at[pl.ds(j0, n)], sem_b))
        for d in descs:
            d.wait()

        @pl.loop(0, _CHUNK)
        def _row(b):
            r0 = b * _F
            ix0 = r0 + iota
            ix1 = ix0 + _L
            vv0 = plsc.load_gather(fv_v, [ix0])
            vv1 = plsc.load_gather(fv_v, [ix1], mask=m1)
            qo0 = (plsc.load_gather(idx_v, [ix0]) & 3) * _D
            qo1 = (plsc.load_gather(idx_v, [ix1], mask=m1) & 3) * _D
            acc0 = zero_v
            acc1 = zero_v
            sq0 = zero_v
            sq1 = zero_v
            rsplat = jnp.full((_L,), 0, jnp.int32) + r0
            for f in range(_F):
                sv = (vv0 if f < _L else vv1)[f % _L]
                co = (qo0 if f < _L else qo1)[f % _L]
                rvec = rsplat + f
                c0 = co + iota
                e0 = plsc.load_gather(emb_v, [rvec, c0])
                e1 = plsc.load_gather(emb_v, [rvec, c0 + _L])
                ev0 = e0 * sv
                ev1 = e1 * sv
                acc0 = acc0 + ev0
                acc1 = acc1 + ev1
                sq0 = sq0 + ev0 * ev0
                sq1 = sq1 + ev1 * ev1
            fm = acc0 * acc0 - sq0 + acc1 * acc1 - sq1
            bb0 = plsc.load_gather(bias_v, [ix0])
            bb1 = plsc.load_gather(bias_v, [ix1], mask=m1)
            bvec = bb0 * vv0 + jnp.where(m1, bb1 * vv1, 0.0)
            t = lax.reduce_sum(0.5 * fm + bvec, axes=(0,))
            tv = jnp.full((_L,), 0.0, jnp.float32) + t
            plsc.store_scatter(out_v, [jnp.full((_L,), 0, jnp.int32) + b],
                               tv, mask=mlast)

        pltpu.sync_copy(out_v, out_hbm.at[pl.ds(wid * _RPW + c * _CHUNK,
                                                _CHUNK)])


_MESH = plsc.VectorSubcoreMesh(core_axis_name="c", subcore_axis_name="s")
_PARAMS = pltpu.CompilerParams(
    needs_layout_passes=False, use_tc_tiling_on_sc=True)


@jax.jit
def _pointfm_sc(feat_flat, fv_flat, emb_t, tail16, bias_flat):
    sup = pl.kernel(
        _tbody,
        out_type=jax.ShapeDtypeStruct((_NSUP, _SROW), jnp.float32),
        mesh=_MESH,
        scratch_types=[
            pltpu.VMEM((2, _D, _W), jnp.float32),
            pltpu.VMEM((2, _W // 4, _SROW), jnp.float32),
            pltpu.SemaphoreType.DMA,
            pltpu.SemaphoreType.DMA,
            pltpu.SemaphoreType.DMA,
            pltpu.SemaphoreType.DMA,
        ],
        compiler_params=_PARAMS,
    )(emb_t, tail16)
    return pl.kernel(
        _gbody,
        out_type=jax.ShapeDtypeStruct((_B,), jnp.float32),
        mesh=_MESH,
        scratch_types=[
            pltpu.VMEM((_IPC,), jnp.int32),
            pltpu.VMEM((_IPC,), jnp.int32),
            pltpu.VMEM((_IPC, _SROW), jnp.float32),
            pltpu.VMEM((_IPC,), jnp.float32),
            pltpu.VMEM((_IPC,), jnp.float32),
            pltpu.VMEM((_CHUNK,), jnp.float32),
            pltpu.SemaphoreType.DMA,
            pltpu.SemaphoreType.DMA,
        ],
        compiler_params=_PARAMS,
    )(feat_flat, fv_flat, sup, bias_flat)


def kernel(features, feature_values, emb_table, bias_table, bias_):
    tail16 = emb_table[_V - 2 * _D:].reshape(16, _SROW)
    out = _pointfm_sc(features.reshape(-1), feature_values.reshape(-1),
                      emb_table.T, tail16, bias_table.reshape(-1))
    return out + bias_
